# baseline (speedup 1.0000x reference)
import functools

import jax
import jax.numpy as jnp
from jax import lax
from jax.experimental import pallas as pl
from jax.experimental.pallas import tpu as pltpu
from jax.experimental.pallas import tpu_sc as plsc

B = 16384
NW, NT, ND = 18, 18, 12
E = 64
H = 256
C = 75

NC, NS = 2, 16
NWORKERS = NC * NS
CH = 128


KB = 4
NSLOT = 3


def _make_sc_gather():
    n_w = B * NW // (NWORKERS * CH)
    n_t = B * NT // (NWORKERS * CH)
    n_d = B * ND // (NWORKERS * CH)
    mesh = plsc.VectorSubcoreMesh(core_axis_name="c", subcore_axis_name="s")

    @functools.partial(
        pl.kernel,
        out_type=[
            jax.ShapeDtypeStruct((B * NW, E), jnp.bfloat16),
            jax.ShapeDtypeStruct((B * NT, E), jnp.bfloat16),
            jax.ShapeDtypeStruct((B * ND, E), jnp.bfloat16),
        ],
        mesh=mesh,
        compiler_params=pltpu.CompilerParams(use_tc_tiling_on_sc=False),
        scratch_types=[
            pltpu.VMEM((n_w * CH,), jnp.int32),
            pltpu.VMEM((n_t * CH,), jnp.int32),
            pltpu.VMEM((n_d * CH,), jnp.int32),
            pltpu.VMEM((NSLOT, KB * CH, E), jnp.bfloat16),
            pltpu.SemaphoreType.DMA,
            pltpu.SemaphoreType.DMA,
        ],
    )
    def sc_gather(wids, tids, dids, wemb, temb, demb, xw, xt, xd,
                  widx_v, tidx_v, didx_v, buf, gsem, ssem):
        wid = lax.axis_index("s") * NC + lax.axis_index("c")

        def run(idx_hbm, idx_v, table, out, n_chunks):
            nblk = n_chunks // KB
            base = wid * n_chunks
            rows = KB * CH
            pltpu.sync_copy(idx_hbm.at[pl.ds(base * CH, n_chunks * CH)], idx_v)

            def g_pair(i):
                slot = lax.rem(i, NSLOT)
                return (table.at[idx_v.at[pl.ds(i * rows, rows)]],
                        buf.at[slot])

            def s_pair(i):
                slot = lax.rem(i, NSLOT)
                return (buf.at[slot],
                        out.at[pl.ds((base + i * KB) * CH, rows)])

            pltpu.async_copy(*g_pair(0), gsem)
            pltpu.async_copy(*g_pair(1), gsem)

            def body(i, carry):
                pltpu.make_async_copy(*g_pair(i), gsem).wait()
                pltpu.async_copy(*s_pair(i), ssem)

                @pl.when(i >= 1)
                def _():
                    pltpu.make_async_copy(*s_pair(i - 1), ssem).wait()

                @pl.when(i + 2 < nblk)
                def _():
                    pltpu.async_copy(*g_pair(i + 2), gsem)

                return carry

            lax.fori_loop(0, nblk, body, 0)
            pltpu.make_async_copy(*s_pair(nblk - 1), ssem).wait()

        run(wids, widx_v, wemb, xw, n_w)
        run(tids, tidx_v, temb, xt, n_t)
        run(dids, didx_v, demb, xd, n_d)

    return sc_gather


def _mlp(xw, xt, xd, W_w, W_t, W_d, b1, U, b2):
    bm = 256
    grid = (B // bm,)

    def body(xw_ref, xt_ref, xd_ref, ww_ref, wt_ref, wd_ref, b1_ref, u_ref,
             b2_ref, o_ref):
        z = jnp.dot(xw_ref[...], ww_ref[...], preferred_element_type=jnp.float32)
        z = z + jnp.dot(xt_ref[...], wt_ref[...], preferred_element_type=jnp.float32)
        z = z + jnp.dot(xd_ref[...], wd_ref[...], preferred_element_type=jnp.float32)
        z = z + b1_ref[...]
        h = jnp.maximum(z, 0.0).astype(jnp.bfloat16)
        o_ref[...] = jnp.dot(h, u_ref[...], preferred_element_type=jnp.float32) + b2_ref[...]

    return pl.pallas_call(
        body,
        grid=grid,
        in_specs=[
            pl.BlockSpec((bm, NW * E), lambda i: (i, 0)),
            pl.BlockSpec((bm, NT * E), lambda i: (i, 0)),
            pl.BlockSpec((bm, ND * E), lambda i: (i, 0)),
            pl.BlockSpec((NW * E, H), lambda i: (0, 0)),
            pl.BlockSpec((NT * E, H), lambda i: (0, 0)),
            pl.BlockSpec((ND * E, H), lambda i: (0, 0)),
            pl.BlockSpec((1, H), lambda i: (0, 0)),
            pl.BlockSpec((H, C), lambda i: (0, 0)),
            pl.BlockSpec((1, C), lambda i: (0, 0)),
        ],
        out_specs=pl.BlockSpec((bm, C), lambda i: (i, 0)),
        out_shape=jax.ShapeDtypeStruct((B, C), jnp.float32),
    )(xw, xt, xd, W_w, W_t, W_d, b1.reshape(1, H), U, b2.reshape(1, C))


def kernel(word_ids, tag_ids, deprel_ids, word_emb, tag_emb, deprel_emb,
           W_w, W_t, W_d, b1, U, b2):
    wids = word_ids.reshape(-1)
    tids = tag_ids.reshape(-1)
    dids = deprel_ids.reshape(-1)
    bf = jnp.bfloat16
    xw, xt, xd = _make_sc_gather()(wids, tids, dids,
                                   word_emb.astype(bf), tag_emb.astype(bf),
                                   deprel_emb.astype(bf))
    return _mlp(xw.reshape(B, NW * E), xt.reshape(B, NT * E),
                xd.reshape(B, ND * E), W_w.astype(bf), W_t.astype(bf),
                W_d.astype(bf), b1, U.astype(bf), b2)


# speedup vs baseline: 3.0089x; 3.0089x over previous
import functools

import jax
import jax.numpy as jnp
from jax import lax
from jax.experimental import pallas as pl
from jax.experimental.pallas import tpu as pltpu
from jax.experimental.pallas import tpu_sc as plsc

B = 16384
NW, NT, ND = 18, 18, 12
E = 64
H = 256
C = 75

NC, NS = 2, 16
NWORKERS = NC * NS
CH = 128
KB = 4
NSLOT = 3


def _make_sc_gather():
    n_w = B * NW // (NWORKERS * CH)
    mesh = plsc.VectorSubcoreMesh(core_axis_name="c", subcore_axis_name="s")

    @functools.partial(
        pl.kernel,
        out_type=jax.ShapeDtypeStruct((B * NW, E), jnp.bfloat16),
        mesh=mesh,
        compiler_params=pltpu.CompilerParams(use_tc_tiling_on_sc=False),
        scratch_types=[
            pltpu.VMEM((n_w * CH,), jnp.int32),
            pltpu.VMEM((NSLOT, KB * CH, E), jnp.bfloat16),
            pltpu.SemaphoreType.DMA,
            pltpu.SemaphoreType.DMA,
        ],
    )
    def sc_gather(wids, wemb, xw, idx_v, buf, gsem, ssem):
        wid = lax.axis_index("s") * NC + lax.axis_index("c")
        nblk = n_w // KB
        base = wid * n_w
        rows = KB * CH
        pltpu.sync_copy(wids.at[pl.ds(base * CH, n_w * CH)], idx_v)

        def g_pair(i):
            slot = lax.rem(i, NSLOT)
            return (wemb.at[idx_v.at[pl.ds(i * rows, rows)]], buf.at[slot])

        def s_pair(i):
            slot = lax.rem(i, NSLOT)
            return (buf.at[slot], xw.at[pl.ds((base + i * KB) * CH, rows)])

        pltpu.async_copy(*g_pair(0), gsem)
        pltpu.async_copy(*g_pair(1), gsem)

        def body(i, carry):
            pltpu.make_async_copy(*g_pair(i), gsem).wait()
            pltpu.async_copy(*s_pair(i), ssem)

            @pl.when(i >= 1)
            def _():
                pltpu.make_async_copy(*s_pair(i - 1), ssem).wait()

            @pl.when(i + 2 < nblk)
            def _():
                pltpu.async_copy(*g_pair(i + 2), gsem)

            return carry

        lax.fori_loop(0, nblk, body, 0)
        pltpu.make_async_copy(*s_pair(nblk - 1), ssem).wait()

    return sc_gather


def _project_tables(tag_embP, deprel_embP, W_t, W_d):
    def body(te_ref, de_ref, wt_ref, wd_ref, tp_ref, dp_ref):
        te = te_ref[...]
        de = de_ref[...]
        for j in range(NT):
            tp_ref[j * E:(j + 1) * E, :] = jnp.dot(
                te, wt_ref[j * E:(j + 1) * E, :],
                preferred_element_type=jnp.float32).astype(jnp.bfloat16)
        for j in range(ND):
            dp_ref[j * E:(j + 1) * E, :] = jnp.dot(
                de, wd_ref[j * E:(j + 1) * E, :],
                preferred_element_type=jnp.float32).astype(jnp.bfloat16)

    return pl.pallas_call(
        body,
        out_shape=[
            jax.ShapeDtypeStruct((NT * E, H), jnp.bfloat16),
            jax.ShapeDtypeStruct((ND * E, H), jnp.bfloat16),
        ],
    )(tag_embP, deprel_embP, W_t, W_d)


def _mlp(xw, tids, dids, W_w, TtP, TdP, b1, U, b2):
    bm = 512
    grid = (B // bm,)

    def onehot_pairs(ids_ref, npos):
        iota = lax.broadcasted_iota(jnp.int32, (bm, 2 * E), 1)
        lane_t = iota & (E - 1)
        hi = iota >= E
        pieces = []
        for jp in range(npos // 2):
            ids = ids_ref[...]
            sel = jnp.where(hi, ids[:, 2 * jp + 1:2 * jp + 2],
                            ids[:, 2 * jp:2 * jp + 1])
            pieces.append((sel == lane_t).astype(jnp.bfloat16))
        return jnp.concatenate(pieces, axis=1)

    def body(xw_ref, tid_ref, did_ref, ww_ref, tp_ref, dp_ref, b1_ref, u_ref,
             b2_ref, o_ref):
        z = jnp.dot(xw_ref[...], ww_ref[...],
                    preferred_element_type=jnp.float32)
        oh_t = onehot_pairs(tid_ref, NT)
        z = z + jnp.dot(oh_t, tp_ref[...], preferred_element_type=jnp.float32)
        oh_d = onehot_pairs(did_ref, ND)
        z = z + jnp.dot(oh_d, dp_ref[...], preferred_element_type=jnp.float32)
        z = z + b1_ref[...]
        h = jnp.maximum(z, 0.0).astype(jnp.bfloat16)
        o_ref[...] = jnp.dot(h, u_ref[...],
                             preferred_element_type=jnp.float32) + b2_ref[...]

    return pl.pallas_call(
        body,
        grid=grid,
        in_specs=[
            pl.BlockSpec((bm, NW * E), lambda i: (i, 0)),
            pl.BlockSpec((bm, NT), lambda i: (i, 0)),
            pl.BlockSpec((bm, ND), lambda i: (i, 0)),
            pl.BlockSpec((NW * E, H), lambda i: (0, 0)),
            pl.BlockSpec((NT * E, H), lambda i: (0, 0)),
            pl.BlockSpec((ND * E, H), lambda i: (0, 0)),
            pl.BlockSpec((1, H), lambda i: (0, 0)),
            pl.BlockSpec((H, C), lambda i: (0, 0)),
            pl.BlockSpec((1, C), lambda i: (0, 0)),
        ],
        out_specs=pl.BlockSpec((bm, C), lambda i: (i, 0)),
        out_shape=jax.ShapeDtypeStruct((B, C), jnp.float32),
    )(xw, tids, dids, W_w, TtP, TdP, b1.reshape(1, H), U, b2.reshape(1, C))


def kernel(word_ids, tag_ids, deprel_ids, word_emb, tag_emb, deprel_emb,
           W_w, W_t, W_d, b1, U, b2):
    bf = jnp.bfloat16
    xw = _make_sc_gather()(word_ids.reshape(-1), word_emb.astype(bf))
    tag_embP = jnp.zeros((E, E), bf).at[:48].set(tag_emb.astype(bf))
    deprel_embP = jnp.zeros((E, E), bf).at[:40].set(deprel_emb.astype(bf))
    TtP, TdP = _project_tables(tag_embP, deprel_embP,
                               W_t.astype(bf), W_d.astype(bf))
    return _mlp(xw.reshape(B, NW * E), tag_ids, deprel_ids,
                W_w.astype(bf), TtP, TdP, b1, U.astype(bf), b2)


# speedup vs baseline: 3.9019x; 1.2968x over previous
import functools

import jax
import jax.numpy as jnp
from jax import lax
from jax.experimental import pallas as pl
from jax.experimental.pallas import tpu as pltpu
from jax.experimental.pallas import tpu_sc as plsc

B = 16384
NW, NT, ND = 18, 18, 12
E = 64
H = 256
C = 75

NC, NS = 2, 16
NWORKERS = NC * NS
NP = NW // 2
BT = B // NWORKERS
GR = 512
NSTEP = BT * NW // GR // 2 * 2
NSLOT = 3


def _make_sc_gather():
    mesh = plsc.VectorSubcoreMesh(core_axis_name="c", subcore_axis_name="s")

    @functools.partial(
        pl.kernel,
        out_type=jax.ShapeDtypeStruct((NP, 2 * B, E), jnp.float32),
        mesh=mesh,
        compiler_params=pltpu.CompilerParams(use_tc_tiling_on_sc=False),
        scratch_types=[
            pltpu.VMEM((BT * NW,), jnp.int32),
            pltpu.VMEM((NSLOT, GR, E), jnp.float32),
            pltpu.SemaphoreType.DMA,
            pltpu.SemaphoreType.DMA,
        ],
    )
    def sc_gather(wids, wemb, xw, idx_v, buf, gsem, ssem):
        wid = lax.axis_index("s") * NC + lax.axis_index("c")
        pltpu.sync_copy(wids.at[pl.ds(wid * BT * NW, BT * NW)], idx_v)

        def g_pair(i, slot):
            return (wemb.at[idx_v.at[pl.ds(i * GR, GR)]], buf.at[slot])

        def s_pair(i, slot):
            jp = i // 2
            cb = lax.rem(i, 2)
            return (buf.at[slot],
                    xw.at[jp, pl.ds(wid * 2 * BT + cb * GR, GR)])

        pltpu.async_copy(*g_pair(0, 0), gsem)
        pltpu.async_copy(*g_pair(1, 1), gsem)

        def outer(g, carry):
            i0 = g * NSLOT
            for p in range(NSLOT):
                i = i0 + p

                pltpu.make_async_copy(*g_pair(i, p), gsem).wait()
                pltpu.async_copy(*s_pair(i, p), ssem)

                @pl.when(i >= 1)
                def _():
                    pltpu.make_async_copy(
                        *s_pair(i - 1, (p - 1) % NSLOT), ssem).wait()

                @pl.when(i + 2 < NSTEP)
                def _():
                    pltpu.async_copy(*g_pair(i + 2, (p + 2) % NSLOT), gsem)

            return carry

        lax.fori_loop(0, NSTEP // NSLOT, outer, 0)
        pltpu.make_async_copy(*s_pair(NSTEP - 1, (NSTEP - 1) % NSLOT),
                              ssem).wait()

    return sc_gather


def _project_tables(tag_embP, deprel_embP, W_t, W_d):
    def body(te_ref, de_ref, wt_ref, wd_ref, tp_ref, dp_ref):
        te = te_ref[...]
        de = de_ref[...]
        for j in range(NT):
            tp_ref[j * E:(j + 1) * E, :] = jnp.dot(
                te, wt_ref[j * E:(j + 1) * E, :],
                preferred_element_type=jnp.float32).astype(jnp.bfloat16)
        for j in range(ND):
            dp_ref[j * E:(j + 1) * E, :] = jnp.dot(
                de, wd_ref[j * E:(j + 1) * E, :],
                preferred_element_type=jnp.float32).astype(jnp.bfloat16)

    return pl.pallas_call(
        body,
        out_shape=[
            jax.ShapeDtypeStruct((NT * E, H), jnp.bfloat16),
            jax.ShapeDtypeStruct((ND * E, H), jnp.bfloat16),
        ],
    )(tag_embP, deprel_embP, W_t, W_d)


def _mlp(xw3, tids, dids, W_w, TtP, TdP, b1, U, b2):
    bm = 512
    grid = (B // bm,)

    def onehot_pairs(ids_ref, npos):
        iota = lax.broadcasted_iota(jnp.int32, (bm, 2 * E), 1)
        lane_t = iota & (E - 1)
        hi = iota >= E
        pieces = []
        for jp in range(npos // 2):
            ids = ids_ref[...]
            sel = jnp.where(hi, ids[:, 2 * jp + 1:2 * jp + 2],
                            ids[:, 2 * jp:2 * jp + 1])
            pieces.append((sel == lane_t).astype(jnp.bfloat16))
        return jnp.concatenate(pieces, axis=1)

    def body(xw_ref, tid_ref, did_ref, ww_ref, tp_ref, dp_ref, b1_ref, u_ref,
             b2_ref, o_ref):
        x3 = xw_ref[...].astype(jnp.bfloat16)
        ww = ww_ref[...]
        z = jnp.dot(x3[0], ww[0:128, :], preferred_element_type=jnp.float32)
        for jp in range(1, NP):
            z = z + jnp.dot(x3[jp], ww[jp * 128:(jp + 1) * 128, :],
                            preferred_element_type=jnp.float32)
        oh_t = onehot_pairs(tid_ref, NT)
        z = z + jnp.dot(oh_t, tp_ref[...], preferred_element_type=jnp.float32)
        oh_d = onehot_pairs(did_ref, ND)
        z = z + jnp.dot(oh_d, dp_ref[...], preferred_element_type=jnp.float32)
        z = z + b1_ref[...]
        h = jnp.maximum(z, 0.0).astype(jnp.bfloat16)
        o_ref[...] = jnp.dot(h, u_ref[...],
                             preferred_element_type=jnp.float32) + b2_ref[...]

    return pl.pallas_call(
        body,
        grid=grid,
        in_specs=[
            pl.BlockSpec((NP, bm, 2 * E), lambda i: (0, i, 0)),
            pl.BlockSpec((bm, NT), lambda i: (i, 0)),
            pl.BlockSpec((bm, ND), lambda i: (i, 0)),
            pl.BlockSpec((NW * E, H), lambda i: (0, 0)),
            pl.BlockSpec((NT * E, H), lambda i: (0, 0)),
            pl.BlockSpec((ND * E, H), lambda i: (0, 0)),
            pl.BlockSpec((1, H), lambda i: (0, 0)),
            pl.BlockSpec((H, C), lambda i: (0, 0)),
            pl.BlockSpec((1, C), lambda i: (0, 0)),
        ],
        out_specs=pl.BlockSpec((bm, C), lambda i: (i, 0)),
        out_shape=jax.ShapeDtypeStruct((B, C), jnp.float32),
    )(xw3, tids, dids, W_w, TtP, TdP, b1.reshape(1, H), U, b2.reshape(1, C))


def kernel(word_ids, tag_ids, deprel_ids, word_emb, tag_emb, deprel_emb,
           W_w, W_t, W_d, b1, U, b2):
    bf = jnp.bfloat16
    wids_perm = (word_ids.reshape(NWORKERS, BT, NP, 2)
                 .transpose(0, 2, 1, 3).reshape(-1))
    xw = _make_sc_gather()(wids_perm, word_emb)
    tag_embP = jnp.zeros((E, E), bf).at[:48].set(tag_emb.astype(bf))
    deprel_embP = jnp.zeros((E, E), bf).at[:40].set(deprel_emb.astype(bf))
    TtP, TdP = _project_tables(tag_embP, deprel_embP,
                               W_t.astype(bf), W_d.astype(bf))
    return _mlp(xw.reshape(NP, B, 2 * E), tag_ids, deprel_ids,
                W_w.astype(bf), TtP, TdP, b1, U.astype(bf), b2)


# speedup vs baseline: 3.9257x; 1.0061x over previous
import functools

import jax
import jax.numpy as jnp
from jax import lax
from jax.experimental import pallas as pl
from jax.experimental.pallas import tpu as pltpu
from jax.experimental.pallas import tpu_sc as plsc

B = 16384
NW, NT, ND = 18, 18, 12
E = 64
H = 256
C = 75

NC, NS = 2, 16
NWORKERS = NC * NS
NP = NW // 2
BT = B // NWORKERS
GR = 512
NSTEP = BT * NW // GR // 2 * 2
NSLOT = 3


def _make_sc_gather():
    mesh = plsc.VectorSubcoreMesh(core_axis_name="c", subcore_axis_name="s")

    @functools.partial(
        pl.kernel,
        out_type=jax.ShapeDtypeStruct((NP, 2 * B, E), jnp.float32),
        mesh=mesh,
        compiler_params=pltpu.CompilerParams(use_tc_tiling_on_sc=False),
        scratch_types=[
            pltpu.VMEM((BT * NW,), jnp.int32),
            pltpu.VMEM((NSLOT, GR, E), jnp.float32),
            pltpu.SemaphoreType.DMA,
            pltpu.SemaphoreType.DMA,
            pltpu.SemaphoreType.DMA,
        ],
    )
    def sc_gather(wids, wemb, xw, idx_v, buf, gsem, ssem, isem):
        wid = lax.axis_index("s") * NC + lax.axis_index("c")
        for jp in range(NP):
            pltpu.async_copy(
                wids.at[pl.ds((jp * B + wid * BT) * 2, 2 * BT)],
                idx_v.at[pl.ds(jp * 2 * BT, 2 * BT)], isem)
        for jp in range(NP):
            pltpu.make_async_copy(
                wids.at[pl.ds((jp * B + wid * BT) * 2, 2 * BT)],
                idx_v.at[pl.ds(jp * 2 * BT, 2 * BT)], isem).wait()

        def g_pair(i, slot):
            return (wemb.at[idx_v.at[pl.ds(i * GR, GR)]], buf.at[slot])

        def s_pair(i, slot):
            jp = i // 2
            cb = lax.rem(i, 2)
            return (buf.at[slot],
                    xw.at[jp, pl.ds(wid * 2 * BT + cb * GR, GR)])

        pltpu.async_copy(*g_pair(0, 0), gsem)
        pltpu.async_copy(*g_pair(1, 1), gsem)

        def outer(g, carry):
            i0 = g * NSLOT
            for p in range(NSLOT):
                i = i0 + p

                pltpu.make_async_copy(*g_pair(i, p), gsem).wait()
                pltpu.async_copy(*s_pair(i, p), ssem)

                @pl.when(i >= 1)
                def _():
                    pltpu.make_async_copy(
                        *s_pair(i - 1, (p - 1) % NSLOT), ssem).wait()

                @pl.when(i + 2 < NSTEP)
                def _():
                    pltpu.async_copy(*g_pair(i + 2, (p + 2) % NSLOT), gsem)

            return carry

        lax.fori_loop(0, NSTEP // NSLOT, outer, 0)
        pltpu.make_async_copy(*s_pair(NSTEP - 1, (NSTEP - 1) % NSLOT),
                              ssem).wait()

    return sc_gather


def _project_tables(tag_embP, deprel_embP, W_t, W_d):
    def body(te_ref, de_ref, wt_ref, wd_ref, tp_ref, dp_ref):
        te = te_ref[...]
        de = de_ref[...]
        for j in range(NT):
            tp_ref[j * E:(j + 1) * E, :] = jnp.dot(
                te, wt_ref[j * E:(j + 1) * E, :],
                preferred_element_type=jnp.float32).astype(jnp.bfloat16)
        for j in range(ND):
            dp_ref[j * E:(j + 1) * E, :] = jnp.dot(
                de, wd_ref[j * E:(j + 1) * E, :],
                preferred_element_type=jnp.float32).astype(jnp.bfloat16)

    return pl.pallas_call(
        body,
        out_shape=[
            jax.ShapeDtypeStruct((NT * E, H), jnp.bfloat16),
            jax.ShapeDtypeStruct((ND * E, H), jnp.bfloat16),
        ],
    )(tag_embP, deprel_embP, W_t, W_d)


def _mlp(xw3, tids, dids, W_w, TtP, TdP, b1, U, b2):
    bm = 512
    grid = (B // bm,)

    def onehot_pairs(ids_ref, npos):
        iota = lax.broadcasted_iota(jnp.int32, (bm, 2 * E), 1)
        lane_t = iota & (E - 1)
        hi = iota >= E
        pieces = []
        for jp in range(npos // 2):
            ids = ids_ref[...]
            sel = jnp.where(hi, ids[:, 2 * jp + 1:2 * jp + 2],
                            ids[:, 2 * jp:2 * jp + 1])
            pieces.append((sel == lane_t).astype(jnp.bfloat16))
        return jnp.concatenate(pieces, axis=1)

    def body(xw_ref, tid_ref, did_ref, ww_ref, tp_ref, dp_ref, b1_ref, u_ref,
             b2_ref, o_ref):
        x3 = xw_ref[...].astype(jnp.bfloat16)
        ww = ww_ref[...]
        z = jnp.dot(x3[0], ww[0:128, :], preferred_element_type=jnp.float32)
        for jp in range(1, NP):
            z = z + jnp.dot(x3[jp], ww[jp * 128:(jp + 1) * 128, :],
                            preferred_element_type=jnp.float32)
        oh_t = onehot_pairs(tid_ref, NT)
        z = z + jnp.dot(oh_t, tp_ref[...], preferred_element_type=jnp.float32)
        oh_d = onehot_pairs(did_ref, ND)
        z = z + jnp.dot(oh_d, dp_ref[...], preferred_element_type=jnp.float32)
        z = z + b1_ref[...]
        h = jnp.maximum(z, 0.0).astype(jnp.bfloat16)
        o_ref[...] = jnp.dot(h, u_ref[...],
                             preferred_element_type=jnp.float32) + b2_ref[...]

    return pl.pallas_call(
        body,
        grid=grid,
        in_specs=[
            pl.BlockSpec((NP, bm, 2 * E), lambda i: (0, i, 0)),
            pl.BlockSpec((bm, NT), lambda i: (i, 0)),
            pl.BlockSpec((bm, ND), lambda i: (i, 0)),
            pl.BlockSpec((NW * E, H), lambda i: (0, 0)),
            pl.BlockSpec((NT * E, H), lambda i: (0, 0)),
            pl.BlockSpec((ND * E, H), lambda i: (0, 0)),
            pl.BlockSpec((1, H), lambda i: (0, 0)),
            pl.BlockSpec((H, C), lambda i: (0, 0)),
            pl.BlockSpec((1, C), lambda i: (0, 0)),
        ],
        out_specs=pl.BlockSpec((bm, C), lambda i: (i, 0)),
        out_shape=jax.ShapeDtypeStruct((B, C), jnp.float32),
    )(xw3, tids, dids, W_w, TtP, TdP, b1.reshape(1, H), U, b2.reshape(1, C))


def kernel(word_ids, tag_ids, deprel_ids, word_emb, tag_emb, deprel_emb,
           W_w, W_t, W_d, b1, U, b2):
    bf = jnp.bfloat16
    wids_perm = jnp.concatenate(
        [word_ids[:, 2 * jp:2 * jp + 2] for jp in range(NP)],
        axis=0).reshape(-1)
    xw = _make_sc_gather()(wids_perm, word_emb)
    tag_embP = jnp.zeros((E, E), bf).at[:48].set(tag_emb.astype(bf))
    deprel_embP = jnp.zeros((E, E), bf).at[:40].set(deprel_emb.astype(bf))
    TtP, TdP = _project_tables(tag_embP, deprel_embP,
                               W_t.astype(bf), W_d.astype(bf))
    return _mlp(xw.reshape(NP, B, 2 * E), tag_ids, deprel_ids,
                W_w.astype(bf), TtP, TdP, b1, U.astype(bf), b2)


# speedup vs baseline: 4.9496x; 1.2608x over previous
import functools

import jax
import jax.numpy as jnp
from jax import lax
from jax.experimental import pallas as pl
from jax.experimental.pallas import tpu as pltpu
from jax.experimental.pallas import tpu_sc as plsc

B = 16384
NW, NT, ND = 18, 18, 12
E = 64
H = 256
C = 75

NC, NS = 2, 16
NWORKERS = NC * NS
BT = B // NWORKERS
GR = BT
NSTEP = NW
NSLOT = 3


def _make_sc_gather():
    mesh = plsc.VectorSubcoreMesh(core_axis_name="c", subcore_axis_name="s")

    @functools.partial(
        pl.kernel,
        out_type=jax.ShapeDtypeStruct((NW, B, E), jnp.float32),
        mesh=mesh,
        compiler_params=pltpu.CompilerParams(use_tc_tiling_on_sc=False),
        scratch_types=[
            pltpu.VMEM((NW * BT,), jnp.int32),
            pltpu.VMEM((NSLOT, GR, E), jnp.float32),
            pltpu.SemaphoreType.DMA,
            pltpu.SemaphoreType.DMA,
            pltpu.SemaphoreType.DMA,
        ],
    )
    def sc_gather(wids, wemb, xw, idx_v, buf, gsem, ssem, isem):
        wid = lax.axis_index("s") * NC + lax.axis_index("c")
        for j in range(NW):
            pltpu.async_copy(
                wids.at[pl.ds(j * B + wid * BT, BT)],
                idx_v.at[pl.ds(j * BT, BT)], isem)
        for j in range(NW):
            pltpu.make_async_copy(
                wids.at[pl.ds(j * B + wid * BT, BT)],
                idx_v.at[pl.ds(j * BT, BT)], isem).wait()

        def g_pair(i, slot):
            return (wemb.at[idx_v.at[pl.ds(i * GR, GR)]], buf.at[slot])

        def s_pair(i, slot):
            return (buf.at[slot], xw.at[i, pl.ds(wid * BT, BT)])

        pltpu.async_copy(*g_pair(0, 0), gsem)
        pltpu.async_copy(*g_pair(1, 1), gsem)

        def outer(g, carry):
            i0 = g * NSLOT
            for p in range(NSLOT):
                i = i0 + p

                pltpu.make_async_copy(*g_pair(i, p), gsem).wait()
                pltpu.async_copy(*s_pair(i, p), ssem)

                @pl.when(i >= 1)
                def _():
                    pltpu.make_async_copy(
                        *s_pair(i - 1, (p - 1) % NSLOT), ssem).wait()

                @pl.when(i + 2 < NSTEP)
                def _():
                    pltpu.async_copy(*g_pair(i + 2, (p + 2) % NSLOT), gsem)

            return carry

        lax.fori_loop(0, NSTEP // NSLOT, outer, 0)
        pltpu.make_async_copy(*s_pair(NSTEP - 1, (NSTEP - 1) % NSLOT),
                              ssem).wait()

    return sc_gather


def _project_tables(tag_embP, deprel_embP, W_t, W_d):
    def body(te_ref, de_ref, wt_ref, wd_ref, tp_ref, dp_ref):
        te = te_ref[...]
        de = de_ref[...]
        for j in range(NT):
            tp_ref[j * E:(j + 1) * E, :] = jnp.dot(
                te, wt_ref[j * E:(j + 1) * E, :],
                preferred_element_type=jnp.float32).astype(jnp.bfloat16)
        for j in range(ND):
            dp_ref[j * E:(j + 1) * E, :] = jnp.dot(
                de, wd_ref[j * E:(j + 1) * E, :],
                preferred_element_type=jnp.float32).astype(jnp.bfloat16)

    return pl.pallas_call(
        body,
        out_shape=[
            jax.ShapeDtypeStruct((NT * E, H), jnp.bfloat16),
            jax.ShapeDtypeStruct((ND * E, H), jnp.bfloat16),
        ],
    )(tag_embP, deprel_embP, W_t, W_d)


def _mlp(xw3, tids, dids, W_w, TtP, TdP, b1, U, b2):
    bm = 512
    grid = (B // bm,)

    def onehot_pairs(ids_ref, npos):
        iota = lax.broadcasted_iota(jnp.int32, (bm, 2 * E), 1)
        lane_t = iota & (E - 1)
        hi = iota >= E
        pieces = []
        for jp in range(npos // 2):
            ids = ids_ref[...]
            sel = jnp.where(hi, ids[:, 2 * jp + 1:2 * jp + 2],
                            ids[:, 2 * jp:2 * jp + 1])
            pieces.append((sel == lane_t).astype(jnp.bfloat16))
        return jnp.concatenate(pieces, axis=1)

    def body(xw_ref, tid_ref, did_ref, ww_ref, tp_ref, dp_ref, b1_ref, u_ref,
             b2_ref, o_ref):
        x3 = xw_ref[...].astype(jnp.bfloat16)
        xE = jnp.concatenate([x3[j, :, :E] for j in range(NW)], axis=1)
        xO = jnp.concatenate([x3[j, :, E:] for j in range(NW)], axis=1)
        ww = ww_ref[...]
        zE = jnp.dot(xE, ww, preferred_element_type=jnp.float32)
        zO = jnp.dot(xO, ww, preferred_element_type=jnp.float32)
        ri = lax.broadcasted_iota(jnp.int32, (bm, bm // 2), 0)
        ci = lax.broadcasted_iota(jnp.int32, (bm, bm // 2), 1)
        pE = (ri == 2 * ci).astype(jnp.bfloat16)
        pO = (ri == 2 * ci + 1).astype(jnp.bfloat16)
        z = (jnp.dot(pE, zE.astype(jnp.bfloat16),
                     preferred_element_type=jnp.float32)
             + jnp.dot(pO, zO.astype(jnp.bfloat16),
                       preferred_element_type=jnp.float32))
        oh_t = onehot_pairs(tid_ref, NT)
        z = z + jnp.dot(oh_t, tp_ref[...], preferred_element_type=jnp.float32)
        oh_d = onehot_pairs(did_ref, ND)
        z = z + jnp.dot(oh_d, dp_ref[...], preferred_element_type=jnp.float32)
        z = z + b1_ref[...]
        h = jnp.maximum(z, 0.0).astype(jnp.bfloat16)
        o_ref[...] = jnp.dot(h, u_ref[...],
                             preferred_element_type=jnp.float32) + b2_ref[...]

    return pl.pallas_call(
        body,
        grid=grid,
        in_specs=[
            pl.BlockSpec((NW, bm // 2, 2 * E), lambda i: (0, i, 0)),
            pl.BlockSpec((bm, NT), lambda i: (i, 0)),
            pl.BlockSpec((bm, ND), lambda i: (i, 0)),
            pl.BlockSpec((NW * E, H), lambda i: (0, 0)),
            pl.BlockSpec((NT * E, H), lambda i: (0, 0)),
            pl.BlockSpec((ND * E, H), lambda i: (0, 0)),
            pl.BlockSpec((1, H), lambda i: (0, 0)),
            pl.BlockSpec((H, C), lambda i: (0, 0)),
            pl.BlockSpec((1, C), lambda i: (0, 0)),
        ],
        out_specs=pl.BlockSpec((bm, C), lambda i: (i, 0)),
        out_shape=jax.ShapeDtypeStruct((B, C), jnp.float32),
    )(xw3, tids, dids, W_w, TtP, TdP, b1.reshape(1, H), U, b2.reshape(1, C))


def kernel(word_ids, tag_ids, deprel_ids, word_emb, tag_emb, deprel_emb,
           W_w, W_t, W_d, b1, U, b2):
    bf = jnp.bfloat16
    xw = _make_sc_gather()(word_ids.T.reshape(-1), word_emb)
    tag_embP = jnp.zeros((E, E), bf).at[:48].set(tag_emb.astype(bf))
    deprel_embP = jnp.zeros((E, E), bf).at[:40].set(deprel_emb.astype(bf))
    TtP, TdP = _project_tables(tag_embP, deprel_embP,
                               W_t.astype(bf), W_d.astype(bf))
    return _mlp(xw.reshape(NW, B // 2, 2 * E), tag_ids, deprel_ids,
                W_w.astype(bf), TtP, TdP, b1, U.astype(bf), b2)


# speedup vs baseline: 5.2143x; 1.0535x over previous
import functools

import jax
import jax.numpy as jnp
from jax import lax
from jax.experimental import pallas as pl
from jax.experimental.pallas import tpu as pltpu
from jax.experimental.pallas import tpu_sc as plsc

B = 16384
NW, NT, ND = 18, 18, 12
E = 64
H = 256
C = 75

NC, NS = 2, 16
NWORKERS = NC * NS
BT = B // NWORKERS
GR = BT
NSTEP = NW
NSLOT = 3


def _make_sc_gather():
    mesh = plsc.VectorSubcoreMesh(core_axis_name="c", subcore_axis_name="s")

    @functools.partial(
        pl.kernel,
        out_type=jax.ShapeDtypeStruct((NW, B, E), jnp.float32),
        mesh=mesh,
        compiler_params=pltpu.CompilerParams(use_tc_tiling_on_sc=False),
        scratch_types=[
            pltpu.VMEM((NW * BT,), jnp.int32),
            pltpu.VMEM((NSLOT, GR, E), jnp.float32),
            pltpu.SemaphoreType.DMA,
            pltpu.SemaphoreType.DMA,
            pltpu.SemaphoreType.DMA,
        ],
    )
    def sc_gather(wids, wemb, xw, idx_v, buf, gsem, ssem, isem):
        wid = lax.axis_index("s") * NC + lax.axis_index("c")
        for j in range(NW):
            pltpu.async_copy(
                wids.at[pl.ds(j * B + wid * BT, BT)],
                idx_v.at[pl.ds(j * BT, BT)], isem)
        for j in range(NW):
            pltpu.make_async_copy(
                wids.at[pl.ds(j * B + wid * BT, BT)],
                idx_v.at[pl.ds(j * BT, BT)], isem).wait()

        def g_pair(i, slot):
            return (wemb.at[idx_v.at[pl.ds(i * GR, GR)]], buf.at[slot])

        def s_pair(i, slot):
            return (buf.at[slot], xw.at[i, pl.ds(wid * BT, BT)])

        pltpu.async_copy(*g_pair(0, 0), gsem)
        pltpu.async_copy(*g_pair(1, 1), gsem)

        def outer(g, carry):
            i0 = g * NSLOT
            for p in range(NSLOT):
                i = i0 + p

                pltpu.make_async_copy(*g_pair(i, p), gsem).wait()
                pltpu.async_copy(*s_pair(i, p), ssem)

                @pl.when(i >= 1)
                def _():
                    pltpu.make_async_copy(
                        *s_pair(i - 1, (p - 1) % NSLOT), ssem).wait()

                @pl.when(i + 2 < NSTEP)
                def _():
                    pltpu.async_copy(*g_pair(i + 2, (p + 2) % NSLOT), gsem)

            return carry

        lax.fori_loop(0, NSTEP // NSLOT, outer, 0)
        pltpu.make_async_copy(*s_pair(NSTEP - 1, (NSTEP - 1) % NSLOT),
                              ssem).wait()

    return sc_gather


def _project_tables(tag_embP, deprel_embP, W_t, W_d, W_w):
    def body(te_ref, de_ref, wt_ref, wd_ref, ww_ref, tp_ref, dp_ref, w2_ref):
        te = te_ref[...]
        de = de_ref[...]
        for j in range(NT):
            tp_ref[j * E:(j + 1) * E, :] = jnp.dot(
                te, wt_ref[j * E:(j + 1) * E, :],
                preferred_element_type=jnp.float32).astype(jnp.bfloat16)
        for j in range(ND):
            dp_ref[j * E:(j + 1) * E, :] = jnp.dot(
                de, wd_ref[j * E:(j + 1) * E, :],
                preferred_element_type=jnp.float32).astype(jnp.bfloat16)
        zero = jnp.zeros((E, H), jnp.bfloat16)
        for j in range(NW):
            wj = ww_ref[j * E:(j + 1) * E, :]
            w2_ref[j * 2 * E:j * 2 * E + E, 0:H] = wj
            w2_ref[j * 2 * E:j * 2 * E + E, H:2 * H] = zero
            w2_ref[j * 2 * E + E:(j + 1) * 2 * E, 0:H] = zero
            w2_ref[j * 2 * E + E:(j + 1) * 2 * E, H:2 * H] = wj

    return pl.pallas_call(
        body,
        out_shape=[
            jax.ShapeDtypeStruct((NT * E, H), jnp.bfloat16),
            jax.ShapeDtypeStruct((ND * E, H), jnp.bfloat16),
            jax.ShapeDtypeStruct((NW * 2 * E, 2 * H), jnp.bfloat16),
        ],
    )(tag_embP, deprel_embP, W_t, W_d, W_w)


def _mlp(xw3, tids, dids, W_w, TtP, TdP, b1, U, b2):
    bm = 512
    grid = (B // bm,)

    def onehot_pairs(ids_ref, npos):
        iota = lax.broadcasted_iota(jnp.int32, (bm, 2 * E), 1)
        lane_t = iota & (E - 1)
        hi = iota >= E
        pieces = []
        for jp in range(npos // 2):
            ids = ids_ref[...]
            sel = jnp.where(hi, ids[:, 2 * jp + 1:2 * jp + 2],
                            ids[:, 2 * jp:2 * jp + 1])
            pieces.append((sel == lane_t).astype(jnp.bfloat16))
        return jnp.concatenate(pieces, axis=1)

    def body(xw_ref, tid_ref, did_ref, w2_ref, tp_ref, dp_ref, b1_ref, u_ref,
             b2_ref, o_ref):
        x3 = xw_ref[...].astype(jnp.bfloat16)
        w2 = w2_ref[...]
        zp = jnp.dot(x3[0], w2[0:2 * E, :], preferred_element_type=jnp.float32)
        for j in range(1, NW):
            zp = zp + jnp.dot(x3[j], w2[j * 2 * E:(j + 1) * 2 * E, :],
                              preferred_element_type=jnp.float32)
        zE = zp[:, :H]
        zO = zp[:, H:]
        ri = lax.broadcasted_iota(jnp.int32, (bm, bm // 2), 0)
        ci = lax.broadcasted_iota(jnp.int32, (bm, bm // 2), 1)
        pE = (ri == 2 * ci).astype(jnp.bfloat16)
        pO = (ri == 2 * ci + 1).astype(jnp.bfloat16)
        z = (jnp.dot(pE, zE.astype(jnp.bfloat16),
                     preferred_element_type=jnp.float32)
             + jnp.dot(pO, zO.astype(jnp.bfloat16),
                       preferred_element_type=jnp.float32))
        oh_t = onehot_pairs(tid_ref, NT)
        z = z + jnp.dot(oh_t, tp_ref[...], preferred_element_type=jnp.float32)
        oh_d = onehot_pairs(did_ref, ND)
        z = z + jnp.dot(oh_d, dp_ref[...], preferred_element_type=jnp.float32)
        z = z + b1_ref[...]
        h = jnp.maximum(z, 0.0).astype(jnp.bfloat16)
        o_ref[...] = jnp.dot(h, u_ref[...],
                             preferred_element_type=jnp.float32) + b2_ref[...]

    return pl.pallas_call(
        body,
        grid=grid,
        in_specs=[
            pl.BlockSpec((NW, bm // 2, 2 * E), lambda i: (0, i, 0)),
            pl.BlockSpec((bm, NT), lambda i: (i, 0)),
            pl.BlockSpec((bm, ND), lambda i: (i, 0)),
            pl.BlockSpec((NW * 2 * E, 2 * H), lambda i: (0, 0)),
            pl.BlockSpec((NT * E, H), lambda i: (0, 0)),
            pl.BlockSpec((ND * E, H), lambda i: (0, 0)),
            pl.BlockSpec((1, H), lambda i: (0, 0)),
            pl.BlockSpec((H, C), lambda i: (0, 0)),
            pl.BlockSpec((1, C), lambda i: (0, 0)),
        ],
        out_specs=pl.BlockSpec((bm, C), lambda i: (i, 0)),
        out_shape=jax.ShapeDtypeStruct((B, C), jnp.float32),
    )(xw3, tids, dids, W_w, TtP, TdP, b1.reshape(1, H), U, b2.reshape(1, C))


def kernel(word_ids, tag_ids, deprel_ids, word_emb, tag_emb, deprel_emb,
           W_w, W_t, W_d, b1, U, b2):
    bf = jnp.bfloat16
    xw = _make_sc_gather()(word_ids.T.reshape(-1), word_emb)
    tag_embP = jnp.zeros((E, E), bf).at[:48].set(tag_emb.astype(bf))
    deprel_embP = jnp.zeros((E, E), bf).at[:40].set(deprel_emb.astype(bf))
    TtP, TdP, W2 = _project_tables(tag_embP, deprel_embP,
                                   W_t.astype(bf), W_d.astype(bf),
                                   W_w.astype(bf))
    return _mlp(xw.reshape(NW, B // 2, 2 * E), tag_ids, deprel_ids,
                W2, TtP, TdP, b1, U.astype(bf), b2)


# speedup vs baseline: 5.3903x; 1.0338x over previous
import functools

import jax
import jax.numpy as jnp
from jax import lax
from jax.experimental import pallas as pl
from jax.experimental.pallas import tpu as pltpu
from jax.experimental.pallas import tpu_sc as plsc

B = 16384
NW, NT, ND = 18, 18, 12
E = 64
H = 256
C = 75

NC, NS = 2, 16
NWORKERS = NC * NS
BT = B // NWORKERS
GR = BT
NSTEP = NW
NSLOT = 3


def _make_sc_gather():
    mesh = plsc.VectorSubcoreMesh(core_axis_name="c", subcore_axis_name="s")

    @functools.partial(
        pl.kernel,
        out_type=jax.ShapeDtypeStruct((NW, B, E), jnp.float32),
        mesh=mesh,
        compiler_params=pltpu.CompilerParams(use_tc_tiling_on_sc=False),
        scratch_types=[
            pltpu.VMEM((NW * BT,), jnp.int32),
            pltpu.VMEM((NSLOT, GR, E), jnp.float32),
            pltpu.SemaphoreType.DMA,
            pltpu.SemaphoreType.DMA,
            pltpu.SemaphoreType.DMA,
        ],
    )
    def sc_gather(wids, wemb, xw, idx_v, buf, gsem, ssem, isem):
        wid = lax.axis_index("s") * NC + lax.axis_index("c")
        for j in range(NW):
            pltpu.async_copy(
                wids.at[pl.ds(j * B + wid * BT, BT)],
                idx_v.at[pl.ds(j * BT, BT)], isem)
        for j in range(NW):
            pltpu.make_async_copy(
                wids.at[pl.ds(j * B + wid * BT, BT)],
                idx_v.at[pl.ds(j * BT, BT)], isem).wait()

        def g_pair(i, slot):
            return (wemb.at[idx_v.at[pl.ds(i * GR, GR)]], buf.at[slot])

        def s_pair(i, slot):
            return (buf.at[slot], xw.at[i, pl.ds(wid * BT, BT)])

        pltpu.async_copy(*g_pair(0, 0), gsem)
        pltpu.async_copy(*g_pair(1, 1), gsem)

        def outer(g, carry):
            i0 = g * NSLOT
            for p in range(NSLOT):
                i = i0 + p

                pltpu.make_async_copy(*g_pair(i, p), gsem).wait()
                pltpu.async_copy(*s_pair(i, p), ssem)

                @pl.when(i >= 1)
                def _():
                    pltpu.make_async_copy(
                        *s_pair(i - 1, (p - 1) % NSLOT), ssem).wait()

                @pl.when(i + 2 < NSTEP)
                def _():
                    pltpu.async_copy(*g_pair(i + 2, (p + 2) % NSLOT), gsem)

            return carry

        lax.fori_loop(0, NSTEP // NSLOT, outer, 0)
        pltpu.make_async_copy(*s_pair(NSTEP - 1, (NSTEP - 1) % NSLOT),
                              ssem).wait()

    return sc_gather


def _project_tables(tag_embP, deprel_embP, W_t, W_d, W_w):
    def body(te_ref, de_ref, wt_ref, wd_ref, ww_ref, tp_ref, dp_ref, w2_ref):
        te = te_ref[...]
        de = de_ref[...]
        for j in range(NT):
            tp_ref[j * E:(j + 1) * E, :] = jnp.dot(
                te, wt_ref[j * E:(j + 1) * E, :],
                preferred_element_type=jnp.float32).astype(jnp.bfloat16)
        for j in range(ND):
            dp_ref[j * E:(j + 1) * E, :] = jnp.dot(
                de, wd_ref[j * E:(j + 1) * E, :],
                preferred_element_type=jnp.float32).astype(jnp.bfloat16)
        zero = jnp.zeros((E, H), jnp.bfloat16)
        for j in range(NW):
            wj = ww_ref[j * E:(j + 1) * E, :]
            w2_ref[j * 2 * E:j * 2 * E + E, 0:H] = wj
            w2_ref[j * 2 * E:j * 2 * E + E, H:2 * H] = zero
            w2_ref[j * 2 * E + E:(j + 1) * 2 * E, 0:H] = zero
            w2_ref[j * 2 * E + E:(j + 1) * 2 * E, H:2 * H] = wj

    return pl.pallas_call(
        body,
        out_shape=[
            jax.ShapeDtypeStruct((NT * E, H), jnp.bfloat16),
            jax.ShapeDtypeStruct((ND * E, H), jnp.bfloat16),
            jax.ShapeDtypeStruct((NW * 2 * E, 2 * H), jnp.bfloat16),
        ],
    )(tag_embP, deprel_embP, W_t, W_d, W_w)


def _mlp(xw3, tids, dids, W_w, TtP, TdP, b1, U, b2):
    bm = 1024
    grid = (B // bm,)

    def onehot_pairs(ids_ref, npos):
        iota = lax.broadcasted_iota(jnp.int32, (bm, 2 * E), 1)
        lane_t = iota & (E - 1)
        hi = iota >= E
        pieces = []
        for jp in range(npos // 2):
            ids = ids_ref[...]
            sel = jnp.where(hi, ids[:, 2 * jp + 1:2 * jp + 2],
                            ids[:, 2 * jp:2 * jp + 1])
            pieces.append((sel == lane_t).astype(jnp.bfloat16))
        return jnp.concatenate(pieces, axis=1)

    def body(xw_ref, tid_ref, did_ref, w2_ref, tp_ref, dp_ref, b1_ref, u_ref,
             b2_ref, o_ref):
        x3 = xw_ref[...].astype(jnp.bfloat16)
        w2 = w2_ref[...]
        zp = jnp.dot(x3[0], w2[0:2 * E, :], preferred_element_type=jnp.float32)
        for j in range(1, NW):
            zp = zp + jnp.dot(x3[j], w2[j * 2 * E:(j + 1) * 2 * E, :],
                              preferred_element_type=jnp.float32)
        zE = zp[:, :H]
        zO = zp[:, H:]
        ri = lax.broadcasted_iota(jnp.int32, (bm, bm // 2), 0)
        ci = lax.broadcasted_iota(jnp.int32, (bm, bm // 2), 1)
        pE = (ri == 2 * ci).astype(jnp.bfloat16)
        pO = (ri == 2 * ci + 1).astype(jnp.bfloat16)
        z = (jnp.dot(pE, zE.astype(jnp.bfloat16),
                     preferred_element_type=jnp.float32)
             + jnp.dot(pO, zO.astype(jnp.bfloat16),
                       preferred_element_type=jnp.float32))
        oh_t = onehot_pairs(tid_ref, NT)
        z = z + jnp.dot(oh_t, tp_ref[...], preferred_element_type=jnp.float32)
        oh_d = onehot_pairs(did_ref, ND)
        z = z + jnp.dot(oh_d, dp_ref[...], preferred_element_type=jnp.float32)
        z = z + b1_ref[...]
        h = jnp.maximum(z, 0.0).astype(jnp.bfloat16)
        o_ref[...] = jnp.dot(h, u_ref[...],
                             preferred_element_type=jnp.float32) + b2_ref[...]

    return pl.pallas_call(
        body,
        grid=grid,
        in_specs=[
            pl.BlockSpec((NW, bm // 2, 2 * E), lambda i: (0, i, 0)),
            pl.BlockSpec((bm, NT), lambda i: (i, 0)),
            pl.BlockSpec((bm, ND), lambda i: (i, 0)),
            pl.BlockSpec((NW * 2 * E, 2 * H), lambda i: (0, 0)),
            pl.BlockSpec((NT * E, H), lambda i: (0, 0)),
            pl.BlockSpec((ND * E, H), lambda i: (0, 0)),
            pl.BlockSpec((1, H), lambda i: (0, 0)),
            pl.BlockSpec((H, C), lambda i: (0, 0)),
            pl.BlockSpec((1, C), lambda i: (0, 0)),
        ],
        out_specs=pl.BlockSpec((bm, C), lambda i: (i, 0)),
        out_shape=jax.ShapeDtypeStruct((B, C), jnp.float32),
    )(xw3, tids, dids, W_w, TtP, TdP, b1.reshape(1, H), U, b2.reshape(1, C))


def kernel(word_ids, tag_ids, deprel_ids, word_emb, tag_emb, deprel_emb,
           W_w, W_t, W_d, b1, U, b2):
    bf = jnp.bfloat16
    xw = _make_sc_gather()(word_ids.T.reshape(-1), word_emb)
    tag_embP = jnp.zeros((E, E), bf).at[:48].set(tag_emb.astype(bf))
    deprel_embP = jnp.zeros((E, E), bf).at[:40].set(deprel_emb.astype(bf))
    TtP, TdP, W2 = _project_tables(tag_embP, deprel_embP,
                                   W_t.astype(bf), W_d.astype(bf),
                                   W_w.astype(bf))
    return _mlp(xw.reshape(NW, B // 2, 2 * E), tag_ids, deprel_ids,
                W2, TtP, TdP, b1, U.astype(bf), b2)
